# baseline (speedup 1.0000x reference)
import functools

import jax
import jax.numpy as jnp
from jax import lax
from jax.experimental import pallas as pl
from jax.experimental.pallas import tpu as pltpu
from jax.experimental.pallas import tpu_sc as plsc

_VOCAB = 1000000
_PRETRAIN_DIM = 64
_EMBED_DIM = 32
_OOV_IDX = 999997
_B = 16384
_L = 50

_G = 4
_FUSE_BLK = 2000


def _fuse_body(pt_ref, id_ref, w4_ref, out_ref):
    i = pl.program_id(0)
    acc = jax.lax.dot_general(
        pt_ref[...], w4_ref[...],
        dimension_numbers=(((1,), (0,)), ((), ())),
        preferred_element_type=jnp.float32,
    ) + id_ref[...]
    row = i * _FUSE_BLK + jax.lax.broadcasted_iota(jnp.int32, (_FUSE_BLK, 128), 0)
    sub = jax.lax.broadcasted_iota(jnp.int32, (_FUSE_BLK, 128), 1) // _EMBED_DIM
    vocab_idx = row * _G + sub
    out_ref[...] = jnp.where(vocab_idx <= _OOV_IDX, acc, 0.0)


def _fuse_tables(pretrain_g, id_g, w4):
    n_rows = _VOCAB // _G
    grid = n_rows // _FUSE_BLK
    return pl.pallas_call(
        _fuse_body,
        grid=(grid,),
        in_specs=[
            pl.BlockSpec((_FUSE_BLK, _G * _PRETRAIN_DIM), lambda i: (i, 0)),
            pl.BlockSpec((_FUSE_BLK, _G * _EMBED_DIM), lambda i: (i, 0)),
            pl.BlockSpec((_G * _PRETRAIN_DIM, _G * _EMBED_DIM), lambda i: (0, 0)),
        ],
        out_specs=pl.BlockSpec((_FUSE_BLK, _G * _EMBED_DIM), lambda i: (i, 0)),
        out_shape=jax.ShapeDtypeStruct((n_rows, _G * _EMBED_DIM), jnp.float32),
    )(pretrain_g, id_g, w4)


_NC, _NS = 2, 16
_NW = _NC * _NS
_RG = 8
_ROWS_PER_W = _B // _NW


def _make_gather():
    n_groups = _ROWS_PER_W // _RG
    mesh = plsc.VectorSubcoreMesh(core_axis_name="c", subcore_axis_name="s")

    @functools.partial(
        pl.kernel,
        mesh=mesh,
        out_type=jax.ShapeDtypeStruct((_B, _L, _EMBED_DIM), jnp.float32),
        scratch_types=[
            pltpu.VMEM((2, _RG, _L), jnp.int32),
            pltpu.VMEM((2, _RG, _L, _EMBED_DIM), jnp.float32),
            pltpu.SemaphoreType.DMA,
            pltpu.SemaphoreType.DMA,
        ],
        compiler_params=pltpu.CompilerParams(use_tc_tiling_on_sc=False),
    )
    def gather_k(table_hbm, idx_hbm, out_hbm, idx_v, rows_v, g_sem, o_sem):
        wid = lax.axis_index("s") * _NC + lax.axis_index("c")
        base = wid * _ROWS_PER_W

        def fire(g, slot):
            b0 = base + g * _RG
            pltpu.sync_copy(idx_hbm.at[pl.ds(b0, _RG)], idx_v.at[slot])
            cps = []
            for j in range(_RG):
                cps.append(pltpu.async_copy(
                    table_hbm.at[idx_v.at[slot].at[j]],
                    rows_v.at[slot].at[j], g_sem))
            return cps

        def drain_store(g, slot, cps):
            for cp in cps:
                cp.wait()
            b0 = base + g * _RG
            return pltpu.async_copy(
                rows_v.at[slot], out_hbm.at[pl.ds(b0, _RG)], o_sem)

        cps = fire(0, 0)
        st = None
        for g in range(1, n_groups):
            slot = g % 2
            nxt = fire(g, slot)
            if st is not None:
                st.wait()
            st = drain_store(g - 1, 1 - slot, cps)
            cps = nxt
        if st is not None:
            st.wait()
        drain_store(n_groups - 1, (n_groups - 1) % 2, cps).wait()

    return gather_k


def kernel(inputs, pretrain_table, id_table, W_proj):
    w4 = jnp.kron(jnp.eye(_G, dtype=jnp.float32), W_proj.T)
    pretrain_g = pretrain_table.reshape(_VOCAB // _G, _G * _PRETRAIN_DIM)
    id_g = id_table.reshape(_VOCAB // _G, _G * _EMBED_DIM)

    fused = _fuse_tables(pretrain_g, id_g, w4).reshape(_VOCAB, _EMBED_DIM)

    return _make_gather()(fused, inputs.astype(jnp.int32))
